# baseline (speedup 1.0000x reference)
import functools

import jax
import jax.numpy as jnp
from jax.experimental import pallas as pl


def _layer_kernel(adj_ref, v_ref, w_ref, b_ref, o_ref, *, act):
    t = jnp.dot(adj_ref[...], v_ref[...], preferred_element_type=jnp.float32)
    o = jnp.dot(t, w_ref[...], preferred_element_type=jnp.float32) + b_ref[...]
    o_ref[...] = act(o)


def _layer(adj, v, w, b, act, block_m):
    n = adj.shape[0]
    d_in = v.shape[1]
    d_out = w.shape[1]
    grid = (n // block_m,)
    return pl.pallas_call(
        functools.partial(_layer_kernel, act=act),
        grid=grid,
        in_specs=[
            pl.BlockSpec((block_m, n), lambda i: (i, 0)),
            pl.BlockSpec((n, d_in), lambda i: (0, 0)),
            pl.BlockSpec((d_in, d_out), lambda i: (0, 0)),
            pl.BlockSpec((1, d_out), lambda i: (0, 0)),
        ],
        out_specs=pl.BlockSpec((block_m, d_out), lambda i: (i, 0)),
        out_shape=jax.ShapeDtypeStruct((n, d_out), jnp.float32),
    )(adj, v, w, b.reshape(1, d_out))


def kernel(x, adj, W1, b1, W4, b4):
    h = _layer(adj, x, W1, b1, jax.nn.relu, block_m=200)
    o = _layer(adj, h, W4, b4, jax.nn.sigmoid, block_m=200)
    return o


# speedup vs baseline: 1.0228x; 1.0228x over previous
import functools

import jax
import jax.numpy as jnp
from jax.experimental import pallas as pl


def _layer_kernel(adj_ref, v_ref, w_ref, b_ref, o_ref, *, act):
    t = jnp.dot(adj_ref[...], v_ref[...], preferred_element_type=jnp.float32)
    o = jnp.dot(t, w_ref[...], preferred_element_type=jnp.float32) + b_ref[...]
    o_ref[...] = act(o)


def _layer(adj, v, w, b, act, block_m):
    n = adj.shape[0]
    d_in = v.shape[1]
    d_out = w.shape[1]
    grid = (n // block_m,)
    return pl.pallas_call(
        functools.partial(_layer_kernel, act=act),
        grid=grid,
        in_specs=[
            pl.BlockSpec((block_m, n), lambda i: (i, 0)),
            pl.BlockSpec((n, d_in), lambda i: (0, 0)),
            pl.BlockSpec((d_in, d_out), lambda i: (0, 0)),
            pl.BlockSpec((1, d_out), lambda i: (0, 0)),
        ],
        out_specs=pl.BlockSpec((block_m, d_out), lambda i: (i, 0)),
        out_shape=jax.ShapeDtypeStruct((n, d_out), jnp.float32),
    )(adj, v, w, b.reshape(1, d_out))


def kernel(x, adj, W1, b1, W4, b4):
    h = _layer(adj, x, W1, b1, jax.nn.relu, block_m=400)
    o = _layer(adj, h, W4, b4, jax.nn.sigmoid, block_m=400)
    return o
